# baseline (speedup 1.0000x reference)
import functools

import jax
import jax.numpy as jnp
from jax import lax
from jax.experimental import pallas as pl
from jax.experimental.pallas import tpu as pltpu
from jax.experimental.pallas import tpu_sc as plsc

B = 4
T = 2048
D = 1024
LANES = 16

_info = plsc.get_sparse_core_info()
NC = _info.num_cores
NS = _info.num_subcores
NW = NC * NS
TPW = T // NW
CH = 16
NCHUNK = TPW // CH
NUNIT = NCHUNK * B


def _sc_kernel(x_hbm, tok_hbm, pos_hbm, out_hbm,
               idx_v, tok_v, pos_v, g0, g1, o0, o1):
    wid = lax.axis_index("s") * NC + lax.axis_index("c")
    t0 = wid * TPW

    gsem = (g0, g1)
    osem = (o0, o1)

    def start_gather(u):
        c, b = divmod(u, B)
        slot = u % 2
        src_row = b * T + t0 + c * CH
        pltpu.sync_copy(x_hbm.at[pl.ds(src_row, CH)], idx_v.at[slot])
        return pltpu.async_copy(tok_hbm.at[idx_v.at[slot]],
                                tok_v.at[slot], gsem[slot])

    gh = {0: start_gather(0)}
    oh = {}

    for u in range(NUNIT):
        c, b = divmod(u, B)
        slot = u % 2

        if u + 1 < NUNIT:
            if u - 1 >= 0:
                oh.pop(u - 1).wait()
            gh[u + 1] = start_gather(u + 1)

        if b == 0:
            pltpu.sync_copy(pos_hbm.at[pl.ds(t0 + c * CH, CH)], pos_v)

        gh.pop(u).wait()

        def body(r, _, slot=slot):
            for k in range(D // LANES):
                vp = pos_v[r, pl.ds(k * LANES, LANES)]
                plsc.addupdate(tok_v.at[slot, r, pl.ds(k * LANES, LANES)], vp)
            return 0

        lax.fori_loop(0, CH, body, 0)

        dst_row = b * T + t0 + c * CH
        oh[u] = pltpu.async_copy(tok_v.at[slot],
                                 out_hbm.at[pl.ds(dst_row, CH)], osem[slot])

    oh.pop(NUNIT - 2).wait()
    oh.pop(NUNIT - 1).wait()


@jax.jit
def kernel(x, token_table, pos_table):
    mesh = plsc.VectorSubcoreMesh(core_axis_name="c", subcore_axis_name="s")
    k = functools.partial(
        pl.kernel,
        mesh=mesh,
        out_type=jax.ShapeDtypeStruct((B * T, D), jnp.float32),
        scratch_types=[
            pltpu.VMEM((2, CH), jnp.int32),
            pltpu.VMEM((2, CH, D), jnp.float32),
            pltpu.VMEM((CH, D), jnp.float32),
            pltpu.SemaphoreType.DMA,
            pltpu.SemaphoreType.DMA,
            pltpu.SemaphoreType.DMA,
            pltpu.SemaphoreType.DMA,
        ],
    )(_sc_kernel)
    out = k(x.reshape(-1), token_table, pos_table)
    return out.reshape(B, T, D)


# speedup vs baseline: 1.0455x; 1.0455x over previous
import functools

import jax
import jax.numpy as jnp
from jax import lax
from jax.experimental import pallas as pl
from jax.experimental.pallas import tpu as pltpu
from jax.experimental.pallas import tpu_sc as plsc

B = 4
T = 2048
D = 1024
LANES = 16

_info = plsc.get_sparse_core_info()
NC = _info.num_cores
NS = _info.num_subcores
NW = NC * NS
TPW = T // NW
CH = 16
NCHUNK = TPW // CH
NUNIT = NCHUNK * B


def _sc_kernel(x_hbm, tok_hbm, pos_hbm, out_hbm,
               idx_v, tok_v, pos_v, g0, g1, o0, o1):
    wid = lax.axis_index("s") * NC + lax.axis_index("c")
    t0 = wid * TPW

    gsem = (g0, g1)
    osem = (o0, o1)

    def start_gather(u):
        c, b = divmod(u, B)
        slot = u % 2
        src_row = b * T + t0 + c * CH
        pltpu.sync_copy(x_hbm.at[pl.ds(src_row, CH)], idx_v.at[slot])
        return pltpu.async_copy(tok_hbm.at[idx_v.at[slot]],
                                tok_v.at[slot], gsem[slot])

    gh = {0: start_gather(0)}
    oh = {}

    for u in range(NUNIT):
        c, b = divmod(u, B)
        slot = u % 2

        if u + 1 < NUNIT:
            if u - 1 >= 0:
                oh.pop(u - 1).wait()
            gh[u + 1] = start_gather(u + 1)

        if b == 0:
            pltpu.sync_copy(pos_hbm.at[pl.ds(t0 + c * CH, CH)], pos_v)

        gh.pop(u).wait()

        @plsc.parallel_loop(0, CH, step=1, unroll=1)
        def _add(r, slot=slot):
            for k in range(D // LANES):
                vp = pos_v[r, pl.ds(k * LANES, LANES)]
                plsc.addupdate(tok_v.at[slot, r, pl.ds(k * LANES, LANES)], vp)

        dst_row = b * T + t0 + c * CH
        oh[u] = pltpu.async_copy(tok_v.at[slot],
                                 out_hbm.at[pl.ds(dst_row, CH)], osem[slot])

    oh.pop(NUNIT - 2).wait()
    oh.pop(NUNIT - 1).wait()


@jax.jit
def kernel(x, token_table, pos_table):
    mesh = plsc.VectorSubcoreMesh(core_axis_name="c", subcore_axis_name="s")
    k = functools.partial(
        pl.kernel,
        mesh=mesh,
        out_type=jax.ShapeDtypeStruct((B * T, D), jnp.float32),
        scratch_types=[
            pltpu.VMEM((2, CH), jnp.int32),
            pltpu.VMEM((2, CH, D), jnp.float32),
            pltpu.VMEM((CH, D), jnp.float32),
            pltpu.SemaphoreType.DMA,
            pltpu.SemaphoreType.DMA,
            pltpu.SemaphoreType.DMA,
            pltpu.SemaphoreType.DMA,
        ],
    )(_sc_kernel)
    out = k(x.reshape(-1), token_table, pos_table)
    return out.reshape(B, T, D)


# speedup vs baseline: 1.3725x; 1.3127x over previous
import functools

import jax
import jax.numpy as jnp
from jax import lax
from jax.experimental import pallas as pl
from jax.experimental.pallas import tpu as pltpu
from jax.experimental.pallas import tpu_sc as plsc

B = 4
T = 2048
D = 1024
LANES = 16

_info = plsc.get_sparse_core_info()
NC = _info.num_cores
NS = _info.num_subcores
NW = NC * NS
TPW = T // NW
CH = 8
NCHUNK = TPW // CH


def _sc_kernel(x_hbm, tok_hbm, pos_hbm, out_hbm,
               idx_v, tok_v, pos_v, g0, g1, p0, p1, o0, o1):
    wid = lax.axis_index("s") * NC + lax.axis_index("c")
    t0 = wid * TPW

    gsem = (g0, g1)
    psem = (p0, p1)
    osem = (o0, o1)

    for b in range(B):
        pltpu.sync_copy(x_hbm.at[pl.ds(b * T + t0, TPW)], idx_v.at[b])

    def start_chunk(c):
        gen = c % 2
        ph = pltpu.async_copy(pos_hbm.at[pl.ds(t0 + c * CH, CH)],
                              pos_v.at[gen], psem[gen])
        ghs = [pltpu.async_copy(tok_hbm.at[idx_v.at[b, pl.ds(c * CH, CH)]],
                                tok_v.at[gen, b], gsem[gen])
               for b in range(B)]
        return ph, ghs

    inflight = {0: start_chunk(0)}
    out_h = {}

    for c in range(NCHUNK):
        gen = c % 2

        if c + 1 < NCHUNK:
            if c - 1 in out_h:
                for h in out_h.pop(c - 1):
                    h.wait()
            inflight[c + 1] = start_chunk(c + 1)

        ph, ghs = inflight.pop(c)
        ph.wait()
        for h in ghs:
            h.wait()

        @plsc.parallel_loop(0, CH, step=1, unroll=1)
        def _add(r, gen=gen):
            for k in range(D // LANES):
                vp = pos_v[gen, r, pl.ds(k * LANES, LANES)]
                for b in range(B):
                    plsc.addupdate(
                        tok_v.at[gen, b, r, pl.ds(k * LANES, LANES)], vp)

        out_h[c] = [
            pltpu.async_copy(tok_v.at[gen, b],
                             out_hbm.at[pl.ds(b * T + t0 + c * CH, CH)],
                             osem[gen])
            for b in range(B)
        ]

    for c in (NCHUNK - 2, NCHUNK - 1):
        for h in out_h.pop(c):
            h.wait()


@jax.jit
def kernel(x, token_table, pos_table):
    mesh = plsc.VectorSubcoreMesh(core_axis_name="c", subcore_axis_name="s")
    k = functools.partial(
        pl.kernel,
        mesh=mesh,
        out_type=jax.ShapeDtypeStruct((B * T, D), jnp.float32),
        scratch_types=[
            pltpu.VMEM((B, TPW), jnp.int32),
            pltpu.VMEM((2, B, CH, D), jnp.float32),
            pltpu.VMEM((2, CH, D), jnp.float32),
            pltpu.SemaphoreType.DMA,
            pltpu.SemaphoreType.DMA,
            pltpu.SemaphoreType.DMA,
            pltpu.SemaphoreType.DMA,
            pltpu.SemaphoreType.DMA,
            pltpu.SemaphoreType.DMA,
        ],
    )(_sc_kernel)
    out = k(x.reshape(-1), token_table, pos_table)
    return out.reshape(B, T, D)


# speedup vs baseline: 1.6287x; 1.1867x over previous
import functools

import jax
import jax.numpy as jnp
from jax import lax
from jax.experimental import pallas as pl
from jax.experimental.pallas import tpu as pltpu
from jax.experimental.pallas import tpu_sc as plsc

B = 4
T = 2048
D = 1024
LANES = 16

_info = plsc.get_sparse_core_info()
NC = _info.num_cores
NS = _info.num_subcores
NW = NC * NS
TPW = T // NW
CH = 8
NCHUNK = TPW // CH
NGEN = 3


def _sc_kernel(x_hbm, tok_hbm, pos_hbm, out_hbm,
               idxc_v, tok_v, pos_v, gsem, psem, osem):
    wid = lax.axis_index("s") * NC + lax.axis_index("c")
    t0 = wid * TPW

    pltpu.sync_copy(x_hbm.at[wid], idxc_v)

    def pos_copy(c, gen):
        return pltpu.make_async_copy(pos_hbm.at[pl.ds(t0 + c * CH, CH)],
                                     pos_v.at[gen], psem.at[gen])

    def tok_copy(c, gen):
        return pltpu.make_async_copy(tok_hbm.at[idxc_v.at[c]],
                                     tok_v.at[gen], gsem.at[gen])

    def out_copy(c, gen, b):
        return pltpu.make_async_copy(tok_v.at[gen, pl.ds(b * CH, CH)],
                                     out_hbm.at[b, pl.ds(t0 + c * CH, CH)],
                                     osem.at[gen])

    def start_chunk(c):
        gen = c % NGEN
        pos_copy(c, gen).start()
        tok_copy(c, gen).start()

    def drain_out(c):
        gen = c % NGEN
        for b in range(B):
            out_copy(c, gen, b).wait()

    start_chunk(0)
    start_chunk(1)

    def chunk_body(c, carry):
        gen = c % NGEN

        @pl.when((c >= 1) & (c + 2 < NCHUNK))
        def _():
            drain_out(c - 1)

        @pl.when(c + 2 < NCHUNK)
        def _():
            start_chunk(c + 2)

        pos_copy(c, gen).wait()
        tok_copy(c, gen).wait()

        @plsc.parallel_loop(0, CH, step=1, unroll=1)
        def _add(r):
            for k in range(D // LANES):
                vp = pos_v[gen, r, pl.ds(k * LANES, LANES)]
                for b in range(B):
                    plsc.addupdate(
                        tok_v.at[gen, b * CH + r, pl.ds(k * LANES, LANES)],
                        vp)

        for b in range(B):
            out_copy(c, gen, b).start()
        return carry

    lax.fori_loop(0, NCHUNK, chunk_body, 0)

    for c in range(NCHUNK - NGEN, NCHUNK):
        drain_out(c)


@jax.jit
def kernel(x, token_table, pos_table):
    mesh = plsc.VectorSubcoreMesh(core_axis_name="c", subcore_axis_name="s")
    k = functools.partial(
        pl.kernel,
        mesh=mesh,
        out_type=jax.ShapeDtypeStruct((B, T, D), jnp.float32),
        scratch_types=[
            pltpu.VMEM((NCHUNK, B * CH), jnp.int32),
            pltpu.VMEM((NGEN, B * CH, D), jnp.float32),
            pltpu.VMEM((NGEN, CH, D), jnp.float32),
            pltpu.SemaphoreType.DMA((NGEN,)),
            pltpu.SemaphoreType.DMA((NGEN,)),
            pltpu.SemaphoreType.DMA((NGEN,)),
        ],
    )(_sc_kernel)
    xc = (x.reshape(B, NW, NCHUNK, CH)
           .transpose(1, 2, 0, 3)
           .reshape(NW, NCHUNK, B * CH))
    return k(xc, token_table, pos_table)
